# Initial kernel scaffold; baseline (speedup 1.0000x reference)
import jax
import jax.numpy as jnp
from jax.experimental import pallas as pl


def kernel(x_trans_node, x_addr_node, edge_index_t2a, edge_index_a2t, Win, b_in, Wk, bk, Wq, bq, Wv, bv, Arel, Mrel, Prel, Wa, ba, skip, Wout, b_out):
    raise NotImplementedError("write your pallas kernel here")


# speedup vs baseline: 12.8679x; 12.8679x over previous
import functools
import math

import jax
import jax.numpy as jnp
import numpy as np
from jax.experimental import pallas as pl
from jax.experimental.pallas import tpu as pltpu

_N = 50000
_E = 256000
_HID = 128
_HEADS = 4
_DH = 32
_L = 2
_OUT = 64

_NB = 2000
_EB = 4000


def _mm_body(x_ref, w_ref, b_ref, o_ref, *, act):
    o = jnp.dot(x_ref[...], w_ref[...], preferred_element_type=jnp.float32)
    o = o + b_ref[...]
    if act == "relu":
        o = jnp.maximum(o, 0.0)
    o_ref[...] = o


def _mm(x, w, b, act="none", block=_NB):
    n, din = x.shape
    dout = w.shape[1]
    return pl.pallas_call(
        functools.partial(_mm_body, act=act),
        grid=(n // block,),
        in_specs=[
            pl.BlockSpec((block, din), lambda i: (i, 0)),
            pl.BlockSpec((din, dout), lambda i: (0, 0)),
            pl.BlockSpec((1, dout), lambda i: (0, 0)),
        ],
        out_specs=pl.BlockSpec((block, dout), lambda i: (i, 0)),
        out_shape=jax.ShapeDtypeStruct((n, dout), jnp.float32),
    )(x, w, b.reshape(1, dout))


def _edge_body(qd_ref, ks_ref, vs_ref, smat_ref, stmat_ref, e_ref, msg_ref):
    s4 = jnp.dot(qd_ref[...] * ks_ref[...], smat_ref[...],
                 preferred_element_type=jnp.float32)
    e4 = jnp.exp(s4)
    e_ref[...] = e4
    msg_ref[...] = vs_ref[...] * jnp.dot(e4, stmat_ref[...],
                                         preferred_element_type=jnp.float32)


def _edge_kernel(qd, ks, vs, smat, stmat):
    e = qd.shape[0]
    return pl.pallas_call(
        _edge_body,
        grid=(e // _EB,),
        in_specs=[
            pl.BlockSpec((_EB, _HID), lambda i: (i, 0)),
            pl.BlockSpec((_EB, _HID), lambda i: (i, 0)),
            pl.BlockSpec((_EB, _HID), lambda i: (i, 0)),
            pl.BlockSpec((_HID, _HEADS), lambda i: (0, 0)),
            pl.BlockSpec((_HEADS, _HID), lambda i: (0, 0)),
        ],
        out_specs=[
            pl.BlockSpec((_EB, _HEADS), lambda i: (i, 0)),
            pl.BlockSpec((_EB, _HID), lambda i: (i, 0)),
        ],
        out_shape=[
            jax.ShapeDtypeStruct((e, _HEADS), jnp.float32),
            jax.ShapeDtypeStruct((e, _HID), jnp.float32),
        ],
    )(qd, ks, vs, smat, stmat)


def _layer_out_body(aggu_ref, z_ref, xold_ref, stmat_ref, wa_ref, ba_ref,
                    alpha_ref, o_ref):
    z128 = jnp.dot(z_ref[...], stmat_ref[...],
                   preferred_element_type=jnp.float32)
    agg = aggu_ref[...] / (z128 + 1e-16)
    g = jax.nn.gelu(agg)
    o = jnp.dot(g, wa_ref[...], preferred_element_type=jnp.float32)
    o = o + ba_ref[...]
    al = alpha_ref[0]
    o_ref[...] = al * o + (1.0 - al) * xold_ref[...]


def _layer_out(agg_u, z, xold, stmat, wa, ba, alpha):
    return pl.pallas_call(
        _layer_out_body,
        grid=(_N // _NB,),
        in_specs=[
            pl.BlockSpec((_NB, _HID), lambda i: (i, 0)),
            pl.BlockSpec((_NB, _HEADS), lambda i: (i, 0)),
            pl.BlockSpec((_NB, _HID), lambda i: (i, 0)),
            pl.BlockSpec((_HEADS, _HID), lambda i: (0, 0)),
            pl.BlockSpec((_HID, _HID), lambda i: (0, 0)),
            pl.BlockSpec((1, _HID), lambda i: (0, 0)),
            pl.BlockSpec(memory_space=pltpu.SMEM),
        ],
        out_specs=pl.BlockSpec((_NB, _HID), lambda i: (i, 0)),
        out_shape=jax.ShapeDtypeStruct((_N, _HID), jnp.float32),
    )(agg_u, z, xold, stmat, wa, ba.reshape(1, _HID), alpha.reshape(1))


def _final_body(x_ref, w_ref, b_ref, o_ref):
    y = jnp.dot(x_ref[...], w_ref[...], preferred_element_type=jnp.float32)
    y = y + b_ref[...]
    y = jnp.where(y >= 0, y, 0.2 * y)
    nrm = jnp.sqrt(jnp.sum(y * y, axis=1, keepdims=True))
    o_ref[...] = y / jnp.maximum(nrm, 1e-12)


def _final(x, w, b):
    return pl.pallas_call(
        _final_body,
        grid=(_N // _NB,),
        in_specs=[
            pl.BlockSpec((_NB, _HID), lambda i: (i, 0)),
            pl.BlockSpec((_HID, _OUT), lambda i: (0, 0)),
            pl.BlockSpec((1, _OUT), lambda i: (0, 0)),
        ],
        out_specs=pl.BlockSpec((_NB, _OUT), lambda i: (i, 0)),
        out_shape=jax.ShapeDtypeStruct((_N, _OUT), jnp.float32),
    )(x, w, b.reshape(1, _OUT))


def _blockdiag(a):
    eye = jnp.eye(_HEADS, dtype=a.dtype)
    m = eye[:, None, :, None] * a[:, :, None, :]
    return m.reshape(_HID, _HID)


def kernel(x_trans_node, x_addr_node, edge_index_t2a, edge_index_a2t, Win,
           b_in, Wk, bk, Wq, bq, Wv, bv, Arel, Mrel, Prel, Wa, ba, skip,
           Wout, b_out):
    sel = (np.arange(_HID)[:, None] // _DH == np.arange(_HEADS)[None, :])
    smat0 = jnp.asarray(sel, dtype=jnp.float32)
    stmat = jnp.asarray(sel.T, dtype=jnp.float32)
    scale = 1.0 / math.sqrt(_DH)

    xs = [_mm(x_trans_node, Win[0], b_in[0], act="relu"),
          _mm(x_addr_node, Win[1], b_in[1], act="relu")]
    eidx = [edge_index_t2a, edge_index_a2t]
    rel = [(0, 1), (1, 0)]

    for l in range(_L):
        proj = []
        for t in (0, 1):
            bd_a = _blockdiag(Arel[l, t])
            bd_m = _blockdiag(Mrel[l, t])
            wcat = jnp.concatenate(
                [Wq[l, t], Wk[l, t] @ bd_a, Wv[l, t] @ bd_m], axis=1)
            bcat = jnp.concatenate([bq[l, t], bk[l, t] @ bd_a, bv[l, t] @ bd_m])
            proj.append(_mm(xs[t], wcat, bcat))

        agg_u = [None, None]
        zs = [None, None]
        for et in range(2):
            st, dt = rel[et]
            src, dst = eidx[et][0], eidx[et][1]
            qd = proj[dt][:, :_HID][dst]
            ks = proj[st][:, _HID:2 * _HID][src]
            vs = proj[st][:, 2 * _HID:][src]
            smat = smat0 * (Prel[l, et] * scale)
            e4, msg = _edge_kernel(qd, ks, vs, smat, stmat)
            zs[dt] = jax.ops.segment_sum(e4, dst, num_segments=_N)
            agg_u[dt] = jax.ops.segment_sum(msg, dst, num_segments=_N)

        new_xs = []
        for t in (0, 1):
            al = jax.nn.sigmoid(skip[l, t])
            new_xs.append(
                _layer_out(agg_u[t], zs[t], xs[t], stmat, Wa[l, t], ba[l, t],
                           al))
        xs = new_xs

    return _final(xs[0], Wout, b_out)


# speedup vs baseline: 32.7177x; 2.5426x over previous
import functools
import math

import jax
import jax.numpy as jnp
import numpy as np
from jax import lax
from jax.experimental import pallas as pl
from jax.experimental.pallas import tpu as pltpu
from jax.experimental.pallas import tpu_sc as plsc

_N = 50000
_E = 256000
_HID = 128
_HEADS = 4
_DH = 32
_L = 2
_OUT = 64
_MW = _HID + _HEADS

_NB = 2000
_EB = 4000

_NW = 32
_PW = _E // _NW
_GC = 80
_NG = _PW // _GC


def _proj_body(x_ref, w_ref, b_ref, q_ref, kv_ref):
    o = jnp.dot(x_ref[...], w_ref[...], preferred_element_type=jnp.float32)
    o = o + b_ref[...]
    q_ref[...] = o[:, :_HID]
    kv_ref[...] = o[:, _HID:]


def _proj(x, w, b):
    n, din = x.shape
    dout = w.shape[1]
    return pl.pallas_call(
        _proj_body,
        grid=(n // _NB,),
        in_specs=[
            pl.BlockSpec((_NB, din), lambda i: (i, 0)),
            pl.BlockSpec((din, dout), lambda i: (0, 0)),
            pl.BlockSpec((1, dout), lambda i: (0, 0)),
        ],
        out_specs=[
            pl.BlockSpec((_NB, _HID), lambda i: (i, 0)),
            pl.BlockSpec((_NB, 2 * _HID), lambda i: (i, 0)),
        ],
        out_shape=[
            jax.ShapeDtypeStruct((n, _HID), jnp.float32),
            jax.ShapeDtypeStruct((n, 2 * _HID), jnp.float32),
        ],
    )(x, w, b.reshape(1, dout))


def _mm_body(x_ref, w_ref, b_ref, o_ref, *, act):
    o = jnp.dot(x_ref[...], w_ref[...], preferred_element_type=jnp.float32)
    o = o + b_ref[...]
    if act == "relu":
        o = jnp.maximum(o, 0.0)
    o_ref[...] = o


def _mm(x, w, b, act="none"):
    n, din = x.shape
    dout = w.shape[1]
    return pl.pallas_call(
        functools.partial(_mm_body, act=act),
        grid=(n // _NB,),
        in_specs=[
            pl.BlockSpec((_NB, din), lambda i: (i, 0)),
            pl.BlockSpec((din, dout), lambda i: (0, 0)),
            pl.BlockSpec((1, dout), lambda i: (0, 0)),
        ],
        out_specs=pl.BlockSpec((_NB, dout), lambda i: (i, 0)),
        out_shape=jax.ShapeDtypeStruct((n, dout), jnp.float32),
    )(x, w, b.reshape(1, dout))


def _edge_body(qd_ref, kv_ref, smat_ref, stmat_ref, msg_ref):
    kv = kv_ref[...]
    ks = kv[:, :_HID]
    vs = kv[:, _HID:]
    s4 = jnp.dot(qd_ref[...] * ks, smat_ref[...],
                 preferred_element_type=jnp.float32)
    e4 = jnp.exp(s4)
    msg = vs * jnp.dot(e4, stmat_ref[...], preferred_element_type=jnp.float32)
    msg_ref[...] = jnp.concatenate([msg, e4], axis=1)


def _edge_kernel(qd, kv, smat, stmat):
    e = qd.shape[0]
    return pl.pallas_call(
        _edge_body,
        grid=(e // _EB,),
        in_specs=[
            pl.BlockSpec((_EB, _HID), lambda i: (i, 0)),
            pl.BlockSpec((_EB, 2 * _HID), lambda i: (i, 0)),
            pl.BlockSpec((_HID, _HEADS), lambda i: (0, 0)),
            pl.BlockSpec((_HEADS, _HID), lambda i: (0, 0)),
        ],
        out_specs=pl.BlockSpec((_EB, _MW), lambda i: (i, 0)),
        out_shape=jax.ShapeDtypeStruct((e, _MW), jnp.float32),
    )(qd, kv, smat, stmat)


def _layer_out_body(agg_ref, xold_ref, stmat_ref, wa_ref, ba_ref, alpha_ref,
                    o_ref):
    agg_ext = agg_ref[...]
    z128 = jnp.dot(agg_ext[:, _HID:], stmat_ref[...],
                   preferred_element_type=jnp.float32)
    agg = agg_ext[:, :_HID] / (z128 + 1e-16)
    g = jax.nn.gelu(agg)
    o = jnp.dot(g, wa_ref[...], preferred_element_type=jnp.float32)
    o = o + ba_ref[...]
    al = alpha_ref[0]
    o_ref[...] = al * o + (1.0 - al) * xold_ref[...]


def _layer_out(agg_ext, xold, stmat, wa, ba, alpha):
    return pl.pallas_call(
        _layer_out_body,
        grid=(_N // _NB,),
        in_specs=[
            pl.BlockSpec((_NB, _MW), lambda i: (i, 0)),
            pl.BlockSpec((_NB, _HID), lambda i: (i, 0)),
            pl.BlockSpec((_HEADS, _HID), lambda i: (0, 0)),
            pl.BlockSpec((_HID, _HID), lambda i: (0, 0)),
            pl.BlockSpec((1, _HID), lambda i: (0, 0)),
            pl.BlockSpec(memory_space=pltpu.SMEM),
        ],
        out_specs=pl.BlockSpec((_NB, _HID), lambda i: (i, 0)),
        out_shape=jax.ShapeDtypeStruct((_N, _HID), jnp.float32),
    )(agg_ext, xold, stmat, wa, ba.reshape(1, _HID), alpha.reshape(1))


def _final_body(x_ref, w_ref, b_ref, o_ref):
    y = jnp.dot(x_ref[...], w_ref[...], preferred_element_type=jnp.float32)
    y = y + b_ref[...]
    y = jnp.where(y >= 0, y, 0.2 * y)
    nrm = jnp.sqrt(jnp.sum(y * y, axis=1, keepdims=True))
    o_ref[...] = y / jnp.maximum(nrm, 1e-12)


def _final(x, w, b):
    return pl.pallas_call(
        _final_body,
        grid=(_N // _NB,),
        in_specs=[
            pl.BlockSpec((_NB, _HID), lambda i: (i, 0)),
            pl.BlockSpec((_HID, _OUT), lambda i: (0, 0)),
            pl.BlockSpec((1, _OUT), lambda i: (0, 0)),
        ],
        out_specs=pl.BlockSpec((_NB, _OUT), lambda i: (i, 0)),
        out_shape=jax.ShapeDtypeStruct((_N, _OUT), jnp.float32),
    )(x, w, b.reshape(1, _OUT))


def _sc_gather_pair_body(qtab, kvtab, dst_h, src_h, qd_out, kv_out,
                         dbuf, sbuf, qrows, kvrows, qsA, ksA, qsB, ksB):
    wid = lax.axis_index("s") * 2 + lax.axis_index("c")
    base = wid * _PW
    pltpu.sync_copy(dst_h.at[pl.ds(base, _PW)], dbuf)
    pltpu.sync_copy(src_h.at[pl.ds(base, _PW)], sbuf)

    def body(j, _):
        g0 = 2 * j
        g1 = 2 * j + 1
        cqa = pltpu.async_copy(qtab.at[dbuf.at[pl.ds(g0 * _GC, _GC)]],
                               qrows.at[0], qsA)
        cka = pltpu.async_copy(kvtab.at[sbuf.at[pl.ds(g0 * _GC, _GC)]],
                               kvrows.at[0], ksA)
        cqb = pltpu.async_copy(qtab.at[dbuf.at[pl.ds(g1 * _GC, _GC)]],
                               qrows.at[1], qsB)
        ckb = pltpu.async_copy(kvtab.at[sbuf.at[pl.ds(g1 * _GC, _GC)]],
                               kvrows.at[1], ksB)
        cqa.wait()
        cka.wait()
        pltpu.sync_copy(qrows.at[0], qd_out.at[pl.ds(base + g0 * _GC, _GC)])
        pltpu.sync_copy(kvrows.at[0], kv_out.at[pl.ds(base + g0 * _GC, _GC)])
        cqb.wait()
        ckb.wait()
        pltpu.sync_copy(qrows.at[1], qd_out.at[pl.ds(base + g1 * _GC, _GC)])
        pltpu.sync_copy(kvrows.at[1], kv_out.at[pl.ds(base + g1 * _GC, _GC)])
        return 0

    lax.fori_loop(0, _NG // 2, body, 0)


def _sc_gather_pair(qtab, kvtab, dst, src):
    mesh = plsc.VectorSubcoreMesh(core_axis_name="c", subcore_axis_name="s")
    k = functools.partial(
        pl.kernel,
        out_type=[
            jax.ShapeDtypeStruct((_E, _HID), jnp.float32),
            jax.ShapeDtypeStruct((_E, 2 * _HID), jnp.float32),
        ],
        mesh=mesh,
        scratch_types=[
            pltpu.VMEM((_PW,), jnp.int32),
            pltpu.VMEM((_PW,), jnp.int32),
            pltpu.VMEM((2, _GC, _HID), jnp.float32),
            pltpu.VMEM((2, _GC, 2 * _HID), jnp.float32),
            pltpu.SemaphoreType.DMA,
            pltpu.SemaphoreType.DMA,
            pltpu.SemaphoreType.DMA,
            pltpu.SemaphoreType.DMA,
        ],
    )(_sc_gather_pair_body)
    return k(qtab, kvtab, dst, src)


def _blockdiag(a):
    eye = jnp.eye(_HEADS, dtype=a.dtype)
    m = eye[:, None, :, None] * a[:, :, None, :]
    return m.reshape(_HID, _HID)


def kernel(x_trans_node, x_addr_node, edge_index_t2a, edge_index_a2t, Win,
           b_in, Wk, bk, Wq, bq, Wv, bv, Arel, Mrel, Prel, Wa, ba, skip,
           Wout, b_out):
    sel = (np.arange(_HID)[:, None] // _DH == np.arange(_HEADS)[None, :])
    smat0 = jnp.asarray(sel, dtype=jnp.float32)
    stmat = jnp.asarray(sel.T, dtype=jnp.float32)
    scale = 1.0 / math.sqrt(_DH)

    xs = [_mm(x_trans_node, Win[0], b_in[0], act="relu"),
          _mm(x_addr_node, Win[1], b_in[1], act="relu")]
    eidx = [edge_index_t2a, edge_index_a2t]
    rel = [(0, 1), (1, 0)]

    for l in range(_L):
        qs, kvs = [], []
        for t in (0, 1):
            bd_a = _blockdiag(Arel[l, t])
            bd_m = _blockdiag(Mrel[l, t])
            wcat = jnp.concatenate(
                [Wq[l, t], Wk[l, t] @ bd_a, Wv[l, t] @ bd_m], axis=1)
            bcat = jnp.concatenate([bq[l, t], bk[l, t] @ bd_a, bv[l, t] @ bd_m])
            q_t, kv_t = _proj(xs[t], wcat, bcat)
            qs.append(q_t)
            kvs.append(kv_t)

        agg_ext = [None, None]
        for et in range(2):
            st, dt = rel[et]
            src, dst = eidx[et][0], eidx[et][1]
            qd, kv = _sc_gather_pair(qs[dt], kvs[st], dst, src)
            smat = smat0 * (Prel[l, et] * scale)
            msg_ext = _edge_kernel(qd, kv, smat, stmat)
            agg_ext[dt] = jax.ops.segment_sum(msg_ext, dst, num_segments=_N)

        new_xs = []
        for t in (0, 1):
            al = jax.nn.sigmoid(skip[l, t])
            new_xs.append(
                _layer_out(agg_ext[t], xs[t], stmat, Wa[l, t], ba[l, t], al))
        xs = new_xs

    return _final(xs[0], Wout, b_out)


# speedup vs baseline: 35.6155x; 1.0886x over previous
import functools
import math

import jax
import jax.numpy as jnp
import numpy as np
from jax import lax
from jax.experimental import pallas as pl
from jax.experimental.pallas import tpu as pltpu
from jax.experimental.pallas import tpu_sc as plsc

_N = 50000
_E = 256000
_HID = 128
_HEADS = 4
_DH = 32
_L = 2
_OUT = 64
_MW = 132

_NB = 2000
_EB = 4000

_NW = 32
_PW = _E // _NW
_GC = 80
_NG = _PW // _GC


def _rne16(x):
    u = lax.bitcast_convert_type(x, jnp.uint32)
    return u + jnp.uint32(0x7FFF) + ((u >> 16) & jnp.uint32(1))


def _proj_body(x_ref, w_ref, b_ref, q_ref, kv_ref):
    o = jnp.dot(x_ref[...], w_ref[...], preferred_element_type=jnp.float32)
    o = o + b_ref[...]
    q_ref[...] = o[:, :_HID]
    kb = _rne16(o[:, _HID:2 * _HID]) & jnp.uint32(0xFFFF0000)
    vb = _rne16(o[:, 2 * _HID:]) >> 16
    kv_ref[...] = lax.bitcast_convert_type(kb | vb, jnp.float32)


def _proj(x, w, b):
    n, din = x.shape
    dout = w.shape[1]
    return pl.pallas_call(
        _proj_body,
        grid=(n // _NB,),
        in_specs=[
            pl.BlockSpec((_NB, din), lambda i: (i, 0)),
            pl.BlockSpec((din, dout), lambda i: (0, 0)),
            pl.BlockSpec((1, dout), lambda i: (0, 0)),
        ],
        out_specs=[
            pl.BlockSpec((_NB, _HID), lambda i: (i, 0)),
            pl.BlockSpec((_NB, _HID), lambda i: (i, 0)),
        ],
        out_shape=[
            jax.ShapeDtypeStruct((n, _HID), jnp.float32),
            jax.ShapeDtypeStruct((n, _HID), jnp.float32),
        ],
    )(x, w, b.reshape(1, dout))


def _mm_body(x_ref, w_ref, b_ref, o_ref, *, act):
    o = jnp.dot(x_ref[...], w_ref[...], preferred_element_type=jnp.float32)
    o = o + b_ref[...]
    if act == "relu":
        o = jnp.maximum(o, 0.0)
    o_ref[...] = o


def _mm(x, w, b, act="none"):
    n, din = x.shape
    dout = w.shape[1]
    return pl.pallas_call(
        functools.partial(_mm_body, act=act),
        grid=(n // _NB,),
        in_specs=[
            pl.BlockSpec((_NB, din), lambda i: (i, 0)),
            pl.BlockSpec((din, dout), lambda i: (0, 0)),
            pl.BlockSpec((1, dout), lambda i: (0, 0)),
        ],
        out_specs=pl.BlockSpec((_NB, dout), lambda i: (i, 0)),
        out_shape=jax.ShapeDtypeStruct((n, dout), jnp.float32),
    )(x, w, b.reshape(1, dout))


def _edge_body(qd_ref, kv_ref, smat_ref, stmat_ref, msg_ref):
    w = lax.bitcast_convert_type(kv_ref[...], jnp.uint32)
    ks = lax.bitcast_convert_type(w & jnp.uint32(0xFFFF0000), jnp.float32)
    vs = lax.bitcast_convert_type(w << 16, jnp.float32)
    s4 = jnp.dot(qd_ref[...] * ks, smat_ref[...],
                 preferred_element_type=jnp.float32)
    e4 = jnp.exp(s4)
    msg = vs * jnp.dot(e4, stmat_ref[...], preferred_element_type=jnp.float32)
    msg_ref[...] = jnp.concatenate([msg, e4], axis=1)


def _edge_kernel(qd, kv, smat, stmat):
    e = qd.shape[0]
    return pl.pallas_call(
        _edge_body,
        grid=(e // _EB,),
        in_specs=[
            pl.BlockSpec((_EB, _HID), lambda i: (i, 0)),
            pl.BlockSpec((_EB, _HID), lambda i: (i, 0)),
            pl.BlockSpec((_HID, _HEADS), lambda i: (0, 0)),
            pl.BlockSpec((_HEADS, _HID), lambda i: (0, 0)),
        ],
        out_specs=pl.BlockSpec((_EB, _MW), lambda i: (i, 0)),
        out_shape=jax.ShapeDtypeStruct((e, _MW), jnp.float32),
    )(qd, kv, smat, stmat)


def _layer_out_body(agg_ref, xold_ref, stmat_ref, wa_ref, ba_ref, alpha_ref,
                    o_ref):
    agg_ext = agg_ref[...]
    z128 = jnp.dot(agg_ext[:, _HID:_HID + _HEADS], stmat_ref[...],
                   preferred_element_type=jnp.float32)
    agg = agg_ext[:, :_HID] / (z128 + 1e-16)
    g = jax.nn.gelu(agg)
    o = jnp.dot(g, wa_ref[...], preferred_element_type=jnp.float32)
    o = o + ba_ref[...]
    al = alpha_ref[0]
    o_ref[...] = al * o + (1.0 - al) * xold_ref[...]


def _layer_out(agg_ext, xold, stmat, wa, ba, alpha):
    return pl.pallas_call(
        _layer_out_body,
        grid=(_N // _NB,),
        in_specs=[
            pl.BlockSpec((_NB, _MW), lambda i: (i, 0)),
            pl.BlockSpec((_NB, _HID), lambda i: (i, 0)),
            pl.BlockSpec((_HEADS, _HID), lambda i: (0, 0)),
            pl.BlockSpec((_HID, _HID), lambda i: (0, 0)),
            pl.BlockSpec((1, _HID), lambda i: (0, 0)),
            pl.BlockSpec(memory_space=pltpu.SMEM),
        ],
        out_specs=pl.BlockSpec((_NB, _HID), lambda i: (i, 0)),
        out_shape=jax.ShapeDtypeStruct((_N, _HID), jnp.float32),
    )(agg_ext, xold, stmat, wa, ba.reshape(1, _HID), alpha.reshape(1))


def _final_body(x_ref, w_ref, b_ref, o_ref):
    y = jnp.dot(x_ref[...], w_ref[...], preferred_element_type=jnp.float32)
    y = y + b_ref[...]
    y = jnp.where(y >= 0, y, 0.2 * y)
    nrm = jnp.sqrt(jnp.sum(y * y, axis=1, keepdims=True))
    o_ref[...] = y / jnp.maximum(nrm, 1e-12)


def _final(x, w, b):
    return pl.pallas_call(
        _final_body,
        grid=(_N // _NB,),
        in_specs=[
            pl.BlockSpec((_NB, _HID), lambda i: (i, 0)),
            pl.BlockSpec((_HID, _OUT), lambda i: (0, 0)),
            pl.BlockSpec((1, _OUT), lambda i: (0, 0)),
        ],
        out_specs=pl.BlockSpec((_NB, _OUT), lambda i: (i, 0)),
        out_shape=jax.ShapeDtypeStruct((_N, _OUT), jnp.float32),
    )(x, w, b.reshape(1, _OUT))


def _sc_gather_pair_body(qtab, kvtab, dst_h, src_h, qd_out, kv_out,
                         dbuf, sbuf, qrows, kvrows, qsA, ksA, qsB, ksB):
    wid = lax.axis_index("s") * 2 + lax.axis_index("c")
    base = wid * _PW
    pltpu.sync_copy(dst_h.at[pl.ds(base, _PW)], dbuf)
    pltpu.sync_copy(src_h.at[pl.ds(base, _PW)], sbuf)

    def body(j, _):
        g0 = 2 * j
        g1 = 2 * j + 1
        cqa = pltpu.async_copy(qtab.at[dbuf.at[pl.ds(g0 * _GC, _GC)]],
                               qrows.at[0], qsA)
        cka = pltpu.async_copy(kvtab.at[sbuf.at[pl.ds(g0 * _GC, _GC)]],
                               kvrows.at[0], ksA)
        cqb = pltpu.async_copy(qtab.at[dbuf.at[pl.ds(g1 * _GC, _GC)]],
                               qrows.at[1], qsB)
        ckb = pltpu.async_copy(kvtab.at[sbuf.at[pl.ds(g1 * _GC, _GC)]],
                               kvrows.at[1], ksB)
        cqa.wait()
        cka.wait()
        pltpu.sync_copy(qrows.at[0], qd_out.at[pl.ds(base + g0 * _GC, _GC)])
        pltpu.sync_copy(kvrows.at[0], kv_out.at[pl.ds(base + g0 * _GC, _GC)])
        cqb.wait()
        ckb.wait()
        pltpu.sync_copy(qrows.at[1], qd_out.at[pl.ds(base + g1 * _GC, _GC)])
        pltpu.sync_copy(kvrows.at[1], kv_out.at[pl.ds(base + g1 * _GC, _GC)])
        return 0

    lax.fori_loop(0, _NG // 2, body, 0)


def _sc_gather_pair(qtab, kvtab, dst, src):
    mesh = plsc.VectorSubcoreMesh(core_axis_name="c", subcore_axis_name="s")
    k = functools.partial(
        pl.kernel,
        out_type=[
            jax.ShapeDtypeStruct((_E, _HID), jnp.float32),
            jax.ShapeDtypeStruct((_E, _HID), jnp.float32),
        ],
        mesh=mesh,
        scratch_types=[
            pltpu.VMEM((_PW,), jnp.int32),
            pltpu.VMEM((_PW,), jnp.int32),
            pltpu.VMEM((2, _GC, _HID), jnp.float32),
            pltpu.VMEM((2, _GC, _HID), jnp.float32),
            pltpu.SemaphoreType.DMA,
            pltpu.SemaphoreType.DMA,
            pltpu.SemaphoreType.DMA,
            pltpu.SemaphoreType.DMA,
        ],
    )(_sc_gather_pair_body)
    return k(qtab, kvtab, dst, src)


def _blockdiag(a):
    eye = jnp.eye(_HEADS, dtype=a.dtype)
    m = eye[:, None, :, None] * a[:, :, None, :]
    return m.reshape(_HID, _HID)


def kernel(x_trans_node, x_addr_node, edge_index_t2a, edge_index_a2t, Win,
           b_in, Wk, bk, Wq, bq, Wv, bv, Arel, Mrel, Prel, Wa, ba, skip,
           Wout, b_out):
    sel = (np.arange(_HID)[:, None] // _DH == np.arange(_HEADS)[None, :])
    smat0 = jnp.asarray(sel, dtype=jnp.float32)
    stmat = jnp.asarray(sel.T, dtype=jnp.float32)
    scale = 1.0 / math.sqrt(_DH)

    xs = [_mm(x_trans_node, Win[0], b_in[0], act="relu"),
          _mm(x_addr_node, Win[1], b_in[1], act="relu")]
    eidx = [edge_index_t2a, edge_index_a2t]
    rel = [(0, 1), (1, 0)]

    for l in range(_L):
        qs, kvs = [], []
        for t in (0, 1):
            bd_a = _blockdiag(Arel[l, t])
            bd_m = _blockdiag(Mrel[l, t])
            wcat = jnp.concatenate(
                [Wq[l, t], Wk[l, t] @ bd_a, Wv[l, t] @ bd_m], axis=1)
            bcat = jnp.concatenate([bq[l, t], bk[l, t] @ bd_a, bv[l, t] @ bd_m])
            q_t, kv_t = _proj(xs[t], wcat, bcat)
            qs.append(q_t)
            kvs.append(kv_t)

        agg_ext = [None, None]
        for et in range(2):
            st, dt = rel[et]
            src, dst = eidx[et][0], eidx[et][1]
            qd, kv = _sc_gather_pair(qs[dt], kvs[st], dst, src)
            smat = smat0 * (Prel[l, et] * scale)
            msg_ext = _edge_kernel(qd, kv, smat, stmat)
            agg_ext[dt] = jax.ops.segment_sum(msg_ext, dst, num_segments=_N)

        new_xs = []
        for t in (0, 1):
            al = jax.nn.sigmoid(skip[l, t])
            new_xs.append(
                _layer_out(agg_ext[t], xs[t], stmat, Wa[l, t], ba[l, t], al))
        xs = new_xs

    return _final(xs[0], Wout, b_out)
